# baseline (speedup 1.0000x reference)
import functools

import jax
import jax.numpy as jnp
from jax.experimental import pallas as pl
from jax.experimental.pallas import tpu as pltpu
from jax.experimental.pallas import tpu_sc as plsc

S = 128
L = 100
EMB = 256
WH = 256
SH = 256
NF = 100

_SC_CORES = 2
_SC_SUBCORES = 16
_SC_WORKERS = _SC_CORES * _SC_SUBCORES


def _sc_gather(table, idx):
  B = idx.shape[0]
  D = table.shape[1]
  b_per_w = B // _SC_WORKERS
  mesh = plsc.VectorSubcoreMesh(core_axis_name="c", subcore_axis_name="s")

  @functools.partial(
      pl.kernel,
      out_type=jax.ShapeDtypeStruct((B, D), jnp.float32),
      mesh=mesh,
      scratch_types=[
          pltpu.VMEM((b_per_w,), jnp.int32),
          pltpu.VMEM((b_per_w, D), jnp.float32),
          pltpu.SemaphoreType.DMA,
      ],
  )
  def gather_kernel(table_hbm, idx_hbm, out_hbm, idx_v, rows_v, sem):
    wid = jax.lax.axis_index("s") * _SC_CORES + jax.lax.axis_index("c")
    base = wid * b_per_w
    pltpu.sync_copy(idx_hbm.at[pl.ds(base, b_per_w)], idx_v)
    pltpu.async_copy(table_hbm.at[idx_v], rows_v, sem).wait()
    pltpu.sync_copy(rows_v, out_hbm.at[pl.ds(base, b_per_w)])

  return gather_kernel(table, idx)


def _lstm_gates(g, c):
  i = g[:, 0 * WH:1 * WH]
  f = g[:, 1 * WH:2 * WH]
  gg = g[:, 2 * WH:3 * WH]
  o = g[:, 3 * WH:4 * WH]
  c_new = jax.nn.sigmoid(f) * c + jax.nn.sigmoid(i) * jnp.tanh(gg)
  h_new = jax.nn.sigmoid(o) * jnp.tanh(c_new)
  return h_new, c_new


def _dot(a, b):
  return jnp.dot(a, b, preferred_element_type=jnp.float32)


_CH = 20
_NCH = L // _CH


def _dense_body(wf_ref, x_ref, wih_f_ref, whh_f_ref, wbf_ref, wih_b_ref,
                whh_b_ref, wbb_ref, swihT_f_ref, swhhT_f_ref, sbf_ref,
                swihT_b_ref, swhhT_b_ref, sbb_ref, cw3t_ref, cb3_ref,
                cw4t_ref, cb4_ref, cw5t_ref, cb5_ref, shift_ref, dA_ref,
                dB_ref, dC3_ref, dC4_ref, dC5_ref, db1_ref, dw2T_ref,
                db2_ref, out_ref, xpfw_ref, xpbw_ref, xpf_ref, xpb_ref,
                sof_ref, sob_ref):
  xv = x_ref[...]
  seq_len = jnp.sum(jnp.sign(xv), axis=1, keepdims=True)

  wih_f = wih_f_ref[...]
  whh_f = whh_f_ref[...]
  wbf = wbf_ref[...]
  wih_b = wih_b_ref[...]
  whh_b = whh_b_ref[...]
  wbb = wbb_ref[...]

  def word_chunk(c, carry):
    xpfw_ref[...] = _dot(wf_ref[c].astype(jnp.bfloat16), wih_f) + wbf
    xpbw_ref[...] = _dot(wf_ref[_NCH - 1 - c].astype(jnp.bfloat16),
                         wih_b) + wbb

    def gate_step(tl, carry2):
      hf, cf, af, hb, cb, ab = carry2
      t = c * _CH + tl
      tb = L - 1 - t
      gf = xpfw_ref[pl.ds(tl * S, S), :] + _dot(hf.astype(jnp.bfloat16),
                                                whh_f)
      gb = xpbw_ref[pl.ds((_CH - 1 - tl) * S, S), :] + _dot(
          hb.astype(jnp.bfloat16), whh_b)
      hf, cf = _lstm_gates(gf, cf)
      hb, cb = _lstm_gates(gb, cb)
      mf = jnp.where(t < seq_len, 1.0, 0.0)
      mb = jnp.where(tb < seq_len, 1.0, 0.0)
      return hf, cf, af + hf * mf, hb, cb, ab + hb * mb

    return jax.lax.fori_loop(0, _CH, gate_step, carry)

  z = jnp.zeros((S, WH), jnp.float32)
  _, _, af, _, _, ab = jax.lax.fori_loop(
      0, _NCH, word_chunk, (z, z, z, z, z, z))

  denom = jnp.maximum(seq_len.astype(jnp.float32), 1.0)
  sent_f = af / denom
  sent_b = ab / denom
  sent = jnp.concatenate([sent_f, sent_b], axis=1).astype(jnp.bfloat16)

  xpf_ref[...] = _dot(sent, swihT_f_ref[...]) + sbf_ref[...]
  xpb_ref[...] = _dot(sent, swihT_b_ref[...]) + sbb_ref[...]
  swhhT_f = swhhT_f_ref[...]
  swhhT_b = swhhT_b_ref[...]

  def sent_step(t, carry):
    hf, cf, hb, cb = carry
    tb = S - 1 - t
    gf = xpf_ref[pl.ds(t, 1), :] + _dot(hf.astype(jnp.bfloat16), swhhT_f)
    gb = xpb_ref[pl.ds(tb, 1), :] + _dot(hb.astype(jnp.bfloat16), swhhT_b)
    hf, cf = _lstm_gates(gf, cf)
    hb, cb = _lstm_gates(gb, cb)
    sof_ref[pl.ds(t, 1), :] = hf
    sob_ref[pl.ds(tb, 1), :] = hb
    return hf, cf, hb, cb

  z1 = jnp.zeros((1, SH), jnp.float32)
  jax.lax.fori_loop(0, S, sent_step, (z1, z1, z1, z1))

  so = jnp.concatenate([sof_ref[...], sob_ref[...]], axis=1)
  doc = jnp.mean(so, axis=0, keepdims=True)
  so16 = so.astype(jnp.bfloat16)

  shift = shift_ref[...]
  rows = jax.lax.broadcasted_iota(jnp.int32, (S, NF), 0)

  def conv_pool(wt_ref, k, bias):
    acc = _dot(so16, wt_ref[k - 1])
    for j in range(k - 2, -1, -1):
      acc = _dot(so16, wt_ref[j]) + _dot(shift, acc.astype(jnp.bfloat16))
    out = jax.nn.relu(acc + bias)
    out = jnp.where(rows < S - k + 1, out, 0.0)
    return jnp.max(out, axis=0, keepdims=True)

  l3 = conv_pool(cw3t_ref, 3, cb3_ref[...])
  l4 = conv_pool(cw4t_ref, 4, cb4_ref[...])
  l5 = conv_pool(cw5t_ref, 5, cb5_ref[...])

  h = jnp.tanh(_dot(so16, dA_ref[...]) +
               _dot(doc.astype(jnp.bfloat16), dB_ref[...]) +
               _dot(l3.astype(jnp.bfloat16), dC3_ref[...]) +
               _dot(l4.astype(jnp.bfloat16), dC4_ref[...]) +
               _dot(l5.astype(jnp.bfloat16), dC5_ref[...]) + db1_ref[...])
  out_ref[...] = jax.nn.sigmoid(
      _dot(h.astype(jnp.bfloat16), dw2T_ref[...]) + db2_ref[...])


def _dense_forward(wf3, xv, dense_args, interpret=False):
  return pl.pallas_call(
      _dense_body,
      out_shape=jax.ShapeDtypeStruct((S, 1), jnp.float32),
      scratch_shapes=[
          pltpu.VMEM((_CH * S, 4 * WH), jnp.float32),
          pltpu.VMEM((_CH * S, 4 * WH), jnp.float32),
          pltpu.VMEM((S, 4 * SH), jnp.float32),
          pltpu.VMEM((S, 4 * SH), jnp.float32),
          pltpu.VMEM((S, SH), jnp.float32),
          pltpu.VMEM((S, SH), jnp.float32),
      ],
      compiler_params=pltpu.CompilerParams(
          vmem_limit_bytes=110 * 1024 * 1024),
      interpret=interpret,
  )(wf3, xv, *dense_args)


def _prep_dense_args(ww_ih_f, ww_hh_f, wb_ih_f, wb_hh_f, ww_ih_b, ww_hh_b,
                     wb_ih_b, wb_hh_b, sw_ih_f, sw_hh_f, sb_ih_f, sb_hh_f,
                     sw_ih_b, sw_hh_b, sb_ih_b, sb_hh_b, cw3, cb3, cw4, cb4,
                     cw5, cb5, dw1, db1, dw2, db2):
  dw1T = dw1.T
  b16 = lambda a: a.astype(jnp.bfloat16)
  return (
      b16(ww_ih_f.T), b16(ww_hh_f.T), (wb_ih_f + wb_hh_f).reshape(1, -1),
      b16(ww_ih_b.T), b16(ww_hh_b.T), (wb_ih_b + wb_hh_b).reshape(1, -1),
      b16(sw_ih_f.T), b16(sw_hh_f.T), (sb_ih_f + sb_hh_f).reshape(1, -1),
      b16(sw_ih_b.T), b16(sw_hh_b.T), (sb_ih_b + sb_hh_b).reshape(1, -1),
      b16(cw3[:, 0].transpose(1, 2, 0)), cb3.reshape(1, -1),
      b16(cw4[:, 0].transpose(1, 2, 0)), cb4.reshape(1, -1),
      b16(cw5[:, 0].transpose(1, 2, 0)), cb5.reshape(1, -1),
      jnp.eye(S, k=1, dtype=jnp.bfloat16),
      b16(dw1T[0:2 * SH]), b16(dw1T[2 * SH:4 * SH]),
      b16(dw1T[4 * SH:4 * SH + NF]), b16(dw1T[4 * SH + NF:4 * SH + 2 * NF]),
      b16(dw1T[4 * SH + 2 * NF:4 * SH + 3 * NF]),
      db1.reshape(1, -1), b16(dw2.T), db2.reshape(1, -1),
  )


def kernel(x, emb, ww_ih_f, ww_hh_f, wb_ih_f, wb_hh_f, ww_ih_b, ww_hh_b,
           wb_ih_b, wb_hh_b, sw_ih_f, sw_hh_f, sb_ih_f, sb_hh_f, sw_ih_b,
           sw_hh_b, sb_ih_b, sb_hh_b, cw3, cb3, cw4, cb4, cw5, cb5,
           dw1, db1, dw2, db2):
  xv = x.astype(jnp.int32)
  idx = xv.T.reshape(-1)
  wf = _sc_gather(emb, idx)
  wf3 = wf.reshape(_NCH, _CH * S, EMB)
  dense_args = _prep_dense_args(
      ww_ih_f, ww_hh_f, wb_ih_f, wb_hh_f, ww_ih_b, ww_hh_b, wb_ih_b,
      wb_hh_b, sw_ih_f, sw_hh_f, sb_ih_f, sb_hh_f, sw_ih_b, sw_hh_b,
      sb_ih_b, sb_hh_b, cw3, cb3, cw4, cb4, cw5, cb5, dw1, db1, dw2, db2)
  return _dense_forward(wf3, xv, dense_args)


# speedup vs baseline: 1.0867x; 1.0867x over previous
import functools

import jax
import jax.numpy as jnp
from jax.experimental import pallas as pl
from jax.experimental.pallas import tpu as pltpu
from jax.experimental.pallas import tpu_sc as plsc

S = 128
L = 100
EMB = 256
WH = 256
SH = 256
NF = 100

_SC_CORES = 2
_SC_SUBCORES = 16
_SC_WORKERS = _SC_CORES * _SC_SUBCORES


def _sc_gather(table, idx):
  B = idx.shape[0]
  D = table.shape[1]
  b_per_w = B // _SC_WORKERS
  mesh = plsc.VectorSubcoreMesh(core_axis_name="c", subcore_axis_name="s")

  @functools.partial(
      pl.kernel,
      out_type=jax.ShapeDtypeStruct((B, D), jnp.float32),
      mesh=mesh,
      scratch_types=[
          pltpu.VMEM((b_per_w,), jnp.int32),
          pltpu.VMEM((b_per_w, D), jnp.float32),
          pltpu.SemaphoreType.DMA,
      ],
  )
  def gather_kernel(table_hbm, idx_hbm, out_hbm, idx_v, rows_v, sem):
    wid = jax.lax.axis_index("s") * _SC_CORES + jax.lax.axis_index("c")
    base = wid * b_per_w
    pltpu.sync_copy(idx_hbm.at[pl.ds(base, b_per_w)], idx_v)
    pltpu.async_copy(table_hbm.at[idx_v], rows_v, sem).wait()
    pltpu.sync_copy(rows_v, out_hbm.at[pl.ds(base, b_per_w)])

  return gather_kernel(table, idx)


def _lstm_gates(g, c):
  i = g[:, 0 * WH:1 * WH]
  f = g[:, 1 * WH:2 * WH]
  gg = g[:, 2 * WH:3 * WH]
  o = g[:, 3 * WH:4 * WH]
  c_new = jax.nn.sigmoid(f) * c + jax.nn.sigmoid(i) * jnp.tanh(gg)
  h_new = jax.nn.sigmoid(o) * jnp.tanh(c_new)
  return h_new, c_new


def _dot(a, b):
  return jnp.dot(a, b, preferred_element_type=jnp.float32)


def _dense_body(wf_ref, x_ref, wih_f_ref, whh_f_ref, wbf_ref, wih_b_ref,
                whh_b_ref, wbb_ref, swihT_f_ref, swhhT_f_ref, sbf_ref,
                swihT_b_ref, swhhT_b_ref, sbb_ref, cw3t_ref, cb3_ref,
                cw4t_ref, cb4_ref, cw5t_ref, cb5_ref, shift_ref, dA_ref,
                dB_ref, dC3_ref, dC4_ref, dC5_ref, db1_ref, dw2T_ref,
                db2_ref, out_ref, xpf_ref, xpb_ref, sof_ref, sob_ref):
  xv = x_ref[...]
  seq_len = jnp.sum(jnp.sign(xv), axis=1, keepdims=True)

  wc_f = jnp.concatenate(
      [wih_f_ref[...], whh_f_ref[...]], axis=0)
  wc_b = jnp.concatenate([wih_b_ref[...], whh_b_ref[...]], axis=0)
  wbf = wbf_ref[...]
  wbb = wbb_ref[...]

  def word_step(t, carry):
    hf, cf, af, hb, cb, ab = carry
    tb = L - 1 - t
    xt_f = wf_ref[t].astype(jnp.bfloat16)
    xt_b = wf_ref[tb].astype(jnp.bfloat16)
    inf = jnp.concatenate([xt_f, hf.astype(jnp.bfloat16)], axis=1)
    inb = jnp.concatenate([xt_b, hb.astype(jnp.bfloat16)], axis=1)
    gf = _dot(inf, wc_f) + wbf
    gb = _dot(inb, wc_b) + wbb
    hf, cf = _lstm_gates(gf, cf)
    hb, cb = _lstm_gates(gb, cb)
    mf = jnp.where(t < seq_len, 1.0, 0.0)
    mb = jnp.where(tb < seq_len, 1.0, 0.0)
    return hf, cf, af + hf * mf, hb, cb, ab + hb * mb

  z = jnp.zeros((S, WH), jnp.float32)
  _, _, af, _, _, ab = jax.lax.fori_loop(
      0, L, word_step, (z, z, z, z, z, z))

  denom = jnp.maximum(seq_len.astype(jnp.float32), 1.0)
  sent_f = af / denom
  sent_b = ab / denom
  sent = jnp.concatenate([sent_f, sent_b], axis=1).astype(jnp.bfloat16)

  xpf_ref[...] = _dot(sent, swihT_f_ref[...]) + sbf_ref[...]
  xpb_ref[...] = _dot(sent, swihT_b_ref[...]) + sbb_ref[...]
  swhhT_f = swhhT_f_ref[...]
  swhhT_b = swhhT_b_ref[...]

  def sent_step(i, carry):
    for u in range(2):
      hf, cf, hb, cb = carry
      t = 2 * i + u
      tb = S - 1 - t
      gf = xpf_ref[pl.ds(t, 1), :] + _dot(hf.astype(jnp.bfloat16), swhhT_f)
      gb = xpb_ref[pl.ds(tb, 1), :] + _dot(hb.astype(jnp.bfloat16), swhhT_b)
      hf, cf = _lstm_gates(gf, cf)
      hb, cb = _lstm_gates(gb, cb)
      sof_ref[pl.ds(t, 1), :] = hf
      sob_ref[pl.ds(tb, 1), :] = hb
      carry = (hf, cf, hb, cb)
    return carry

  z1 = jnp.zeros((1, SH), jnp.float32)
  jax.lax.fori_loop(0, S // 2, sent_step, (z1, z1, z1, z1))

  so = jnp.concatenate([sof_ref[...], sob_ref[...]], axis=1)
  doc = jnp.mean(so, axis=0, keepdims=True)
  so16 = so.astype(jnp.bfloat16)

  shift = shift_ref[...]
  rows = jax.lax.broadcasted_iota(jnp.int32, (S, NF), 0)

  def conv_pool(wt_ref, k, bias):
    acc = _dot(so16, wt_ref[k - 1])
    for j in range(k - 2, -1, -1):
      acc = _dot(so16, wt_ref[j]) + _dot(shift, acc.astype(jnp.bfloat16))
    out = jax.nn.relu(acc + bias)
    out = jnp.where(rows < S - k + 1, out, 0.0)
    return jnp.max(out, axis=0, keepdims=True)

  l3 = conv_pool(cw3t_ref, 3, cb3_ref[...])
  l4 = conv_pool(cw4t_ref, 4, cb4_ref[...])
  l5 = conv_pool(cw5t_ref, 5, cb5_ref[...])

  h = jnp.tanh(_dot(so16, dA_ref[...]) +
               _dot(doc.astype(jnp.bfloat16), dB_ref[...]) +
               _dot(l3.astype(jnp.bfloat16), dC3_ref[...]) +
               _dot(l4.astype(jnp.bfloat16), dC4_ref[...]) +
               _dot(l5.astype(jnp.bfloat16), dC5_ref[...]) + db1_ref[...])
  out_ref[...] = jax.nn.sigmoid(
      _dot(h.astype(jnp.bfloat16), dw2T_ref[...]) + db2_ref[...])


def _dense_forward(wf3, xv, dense_args, interpret=False):
  return pl.pallas_call(
      _dense_body,
      out_shape=jax.ShapeDtypeStruct((S, 1), jnp.float32),
      scratch_shapes=[
          pltpu.VMEM((S, 4 * SH), jnp.float32),
          pltpu.VMEM((S, 4 * SH), jnp.float32),
          pltpu.VMEM((S, SH), jnp.float32),
          pltpu.VMEM((S, SH), jnp.float32),
      ],
      compiler_params=pltpu.CompilerParams(
          vmem_limit_bytes=110 * 1024 * 1024),
      interpret=interpret,
  )(wf3, xv, *dense_args)


def _prep_dense_args(ww_ih_f, ww_hh_f, wb_ih_f, wb_hh_f, ww_ih_b, ww_hh_b,
                     wb_ih_b, wb_hh_b, sw_ih_f, sw_hh_f, sb_ih_f, sb_hh_f,
                     sw_ih_b, sw_hh_b, sb_ih_b, sb_hh_b, cw3, cb3, cw4, cb4,
                     cw5, cb5, dw1, db1, dw2, db2):
  dw1T = dw1.T
  b16 = lambda a: a.astype(jnp.bfloat16)
  return (
      b16(ww_ih_f.T), b16(ww_hh_f.T), (wb_ih_f + wb_hh_f).reshape(1, -1),
      b16(ww_ih_b.T), b16(ww_hh_b.T), (wb_ih_b + wb_hh_b).reshape(1, -1),
      b16(sw_ih_f.T), b16(sw_hh_f.T), (sb_ih_f + sb_hh_f).reshape(1, -1),
      b16(sw_ih_b.T), b16(sw_hh_b.T), (sb_ih_b + sb_hh_b).reshape(1, -1),
      b16(cw3[:, 0].transpose(1, 2, 0)), cb3.reshape(1, -1),
      b16(cw4[:, 0].transpose(1, 2, 0)), cb4.reshape(1, -1),
      b16(cw5[:, 0].transpose(1, 2, 0)), cb5.reshape(1, -1),
      jnp.eye(S, k=1, dtype=jnp.bfloat16),
      b16(dw1T[0:2 * SH]), b16(dw1T[2 * SH:4 * SH]),
      b16(dw1T[4 * SH:4 * SH + NF]), b16(dw1T[4 * SH + NF:4 * SH + 2 * NF]),
      b16(dw1T[4 * SH + 2 * NF:4 * SH + 3 * NF]),
      db1.reshape(1, -1), b16(dw2.T), db2.reshape(1, -1),
  )


def kernel(x, emb, ww_ih_f, ww_hh_f, wb_ih_f, wb_hh_f, ww_ih_b, ww_hh_b,
           wb_ih_b, wb_hh_b, sw_ih_f, sw_hh_f, sb_ih_f, sb_hh_f, sw_ih_b,
           sw_hh_b, sb_ih_b, sb_hh_b, cw3, cb3, cw4, cb4, cw5, cb5,
           dw1, db1, dw2, db2):
  xv = x.astype(jnp.int32)
  idx = xv.T.reshape(-1)
  wf = _sc_gather(emb, idx)
  wf3 = wf.reshape(L, S, EMB)
  dense_args = _prep_dense_args(
      ww_ih_f, ww_hh_f, wb_ih_f, wb_hh_f, ww_ih_b, ww_hh_b, wb_ih_b,
      wb_hh_b, sw_ih_f, sw_hh_f, sb_ih_f, sb_hh_f, sw_ih_b, sw_hh_b,
      sb_ih_b, sb_hh_b, cw3, cb3, cw4, cb4, cw5, cb5, dw1, db1, dw2, db2)
  return _dense_forward(wf3, xv, dense_args)


# speedup vs baseline: 1.0901x; 1.0032x over previous
import functools

import jax
import jax.numpy as jnp
from jax.experimental import pallas as pl
from jax.experimental.pallas import tpu as pltpu
from jax.experimental.pallas import tpu_sc as plsc

S = 128
L = 100
EMB = 256
WH = 256
SH = 256
NF = 100

_SC_CORES = 2
_SC_SUBCORES = 16
_SC_WORKERS = _SC_CORES * _SC_SUBCORES


def _sc_gather(table, idx):
  B = idx.shape[0]
  D = table.shape[1]
  b_per_w = B // _SC_WORKERS
  mesh = plsc.VectorSubcoreMesh(core_axis_name="c", subcore_axis_name="s")

  @functools.partial(
      pl.kernel,
      out_type=jax.ShapeDtypeStruct((B, D), jnp.float32),
      mesh=mesh,
      scratch_types=[
          pltpu.VMEM((b_per_w,), jnp.int32),
          pltpu.VMEM((b_per_w, D), jnp.float32),
          pltpu.SemaphoreType.DMA,
      ],
  )
  def gather_kernel(table_hbm, idx_hbm, out_hbm, idx_v, rows_v, sem):
    wid = jax.lax.axis_index("s") * _SC_CORES + jax.lax.axis_index("c")
    base = wid * b_per_w
    pltpu.sync_copy(idx_hbm.at[pl.ds(base, b_per_w)], idx_v)
    pltpu.async_copy(table_hbm.at[idx_v], rows_v, sem).wait()
    pltpu.sync_copy(rows_v, out_hbm.at[pl.ds(base, b_per_w)])

  return gather_kernel(table, idx)


def _lstm_gates(g, c):
  i = g[:, 0 * WH:1 * WH]
  f = g[:, 1 * WH:2 * WH]
  gg = g[:, 2 * WH:3 * WH]
  o = g[:, 3 * WH:4 * WH]
  c_new = jax.nn.sigmoid(f) * c + jax.nn.sigmoid(i) * jnp.tanh(gg)
  h_new = jax.nn.sigmoid(o) * jnp.tanh(c_new)
  return h_new, c_new


def _dot(a, b):
  return jnp.dot(a, b, preferred_element_type=jnp.float32)


_CB = 10
_NB = L // _CB


def _dense_body(wf_ref, x_ref, wih_f_ref, whh_f_ref, wbf_ref, wih_b_ref,
                whh_b_ref, wbb_ref, swihT_f_ref, swhhT_f_ref, sbf_ref,
                swihT_b_ref, swhhT_b_ref, sbb_ref, cw3t_ref, cb3_ref,
                cw4t_ref, cb4_ref, cw5t_ref, cb5_ref, shift_ref, dA_ref,
                dB_ref, dC3_ref, dC4_ref, dC5_ref, db1_ref, dw2T_ref,
                db2_ref, out_ref, buf_ref, sem_ref, xpf_ref, xpb_ref,
                sof_ref, sob_ref):
  xv = x_ref[...]
  seq_len = jnp.sum(jnp.sign(xv), axis=1, keepdims=True)

  wc_f = jnp.concatenate(
      [wih_f_ref[...], whh_f_ref[...]], axis=0)
  wc_b = jnp.concatenate([wih_b_ref[...], whh_b_ref[...]], axis=0)
  wbf = wbf_ref[...]
  wbb = wbb_ref[...]

  def wf_copy(chunk, slot):
    return pltpu.make_async_copy(
        wf_ref.at[pl.ds(chunk * _CB, _CB)], buf_ref.at[slot],
        sem_ref.at[slot])

  wf_copy(0, 0).start()
  wf_copy(_NB - 1, 2).start()
  wf_copy(1, 1).start()
  wf_copy(_NB - 2, 3).start()

  def word_chunk(c, carry):
    slot_f = jax.lax.rem(c, 2)
    slot_b = 2 + slot_f
    wf_copy(c, slot_f).wait()
    wf_copy(_NB - 1 - c, slot_b).wait()

    def gate_step(tl, carry2):
      hf, cf, af, hb, cb, ab = carry2
      t = c * _CB + tl
      tb = L - 1 - t
      xt_f = buf_ref[slot_f, tl].astype(jnp.bfloat16)
      xt_b = buf_ref[slot_b, _CB - 1 - tl].astype(jnp.bfloat16)
      inf = jnp.concatenate([xt_f, hf.astype(jnp.bfloat16)], axis=1)
      inb = jnp.concatenate([xt_b, hb.astype(jnp.bfloat16)], axis=1)
      gf = _dot(inf, wc_f) + wbf
      gb = _dot(inb, wc_b) + wbb
      hf, cf = _lstm_gates(gf, cf)
      hb, cb = _lstm_gates(gb, cb)
      mf = jnp.where(t < seq_len, 1.0, 0.0)
      mb = jnp.where(tb < seq_len, 1.0, 0.0)
      return hf, cf, af + hf * mf, hb, cb, ab + hb * mb

    carry = jax.lax.fori_loop(0, _CB, gate_step, carry)

    @pl.when(c + 2 < _NB)
    def _():
      wf_copy(c + 2, slot_f).start()
      wf_copy(_NB - 3 - c, slot_b).start()

    return carry

  z = jnp.zeros((S, WH), jnp.float32)
  _, _, af, _, _, ab = jax.lax.fori_loop(
      0, _NB, word_chunk, (z, z, z, z, z, z))

  denom = jnp.maximum(seq_len.astype(jnp.float32), 1.0)
  sent_f = af / denom
  sent_b = ab / denom
  sent = jnp.concatenate([sent_f, sent_b], axis=1).astype(jnp.bfloat16)

  xpf_ref[...] = _dot(sent, swihT_f_ref[...]) + sbf_ref[...]
  xpb_ref[...] = _dot(sent, swihT_b_ref[...]) + sbb_ref[...]
  swhhT_f = swhhT_f_ref[...]
  swhhT_b = swhhT_b_ref[...]

  def sent_step(i, carry):
    for u in range(2):
      hf, cf, hb, cb = carry
      t = 2 * i + u
      tb = S - 1 - t
      gf = xpf_ref[pl.ds(t, 1), :] + _dot(hf.astype(jnp.bfloat16), swhhT_f)
      gb = xpb_ref[pl.ds(tb, 1), :] + _dot(hb.astype(jnp.bfloat16), swhhT_b)
      hf, cf = _lstm_gates(gf, cf)
      hb, cb = _lstm_gates(gb, cb)
      sof_ref[pl.ds(t, 1), :] = hf
      sob_ref[pl.ds(tb, 1), :] = hb
      carry = (hf, cf, hb, cb)
    return carry

  z1 = jnp.zeros((1, SH), jnp.float32)
  jax.lax.fori_loop(0, S // 2, sent_step, (z1, z1, z1, z1))

  so = jnp.concatenate([sof_ref[...], sob_ref[...]], axis=1)
  doc = jnp.mean(so, axis=0, keepdims=True)
  so16 = so.astype(jnp.bfloat16)

  shift = shift_ref[...]
  rows = jax.lax.broadcasted_iota(jnp.int32, (S, NF), 0)

  def conv_pool(wt_ref, k, bias):
    acc = _dot(so16, wt_ref[k - 1])
    for j in range(k - 2, -1, -1):
      acc = _dot(so16, wt_ref[j]) + _dot(shift, acc.astype(jnp.bfloat16))
    out = jax.nn.relu(acc + bias)
    out = jnp.where(rows < S - k + 1, out, 0.0)
    return jnp.max(out, axis=0, keepdims=True)

  l3 = conv_pool(cw3t_ref, 3, cb3_ref[...])
  l4 = conv_pool(cw4t_ref, 4, cb4_ref[...])
  l5 = conv_pool(cw5t_ref, 5, cb5_ref[...])

  h = jnp.tanh(_dot(so16, dA_ref[...]) +
               _dot(doc.astype(jnp.bfloat16), dB_ref[...]) +
               _dot(l3.astype(jnp.bfloat16), dC3_ref[...]) +
               _dot(l4.astype(jnp.bfloat16), dC4_ref[...]) +
               _dot(l5.astype(jnp.bfloat16), dC5_ref[...]) + db1_ref[...])
  out_ref[...] = jax.nn.sigmoid(
      _dot(h.astype(jnp.bfloat16), dw2T_ref[...]) + db2_ref[...])


def _dense_forward(wf3, xv, dense_args, interpret=False):
  return pl.pallas_call(
      _dense_body,
      out_shape=jax.ShapeDtypeStruct((S, 1), jnp.float32),
      in_specs=[pl.BlockSpec(memory_space=pltpu.MemorySpace.HBM)] +
               [pl.BlockSpec(memory_space=pltpu.MemorySpace.VMEM)] *
               (1 + len(dense_args)),
      scratch_shapes=[
          pltpu.VMEM((4, _CB, S, EMB), jnp.float32),
          pltpu.SemaphoreType.DMA((4,)),
          pltpu.VMEM((S, 4 * SH), jnp.float32),
          pltpu.VMEM((S, 4 * SH), jnp.float32),
          pltpu.VMEM((S, SH), jnp.float32),
          pltpu.VMEM((S, SH), jnp.float32),
      ],
      compiler_params=pltpu.CompilerParams(
          vmem_limit_bytes=110 * 1024 * 1024),
      interpret=interpret,
  )(wf3, xv, *dense_args)


def _prep_dense_args(ww_ih_f, ww_hh_f, wb_ih_f, wb_hh_f, ww_ih_b, ww_hh_b,
                     wb_ih_b, wb_hh_b, sw_ih_f, sw_hh_f, sb_ih_f, sb_hh_f,
                     sw_ih_b, sw_hh_b, sb_ih_b, sb_hh_b, cw3, cb3, cw4, cb4,
                     cw5, cb5, dw1, db1, dw2, db2):
  dw1T = dw1.T
  b16 = lambda a: a.astype(jnp.bfloat16)
  return (
      b16(ww_ih_f.T), b16(ww_hh_f.T), (wb_ih_f + wb_hh_f).reshape(1, -1),
      b16(ww_ih_b.T), b16(ww_hh_b.T), (wb_ih_b + wb_hh_b).reshape(1, -1),
      b16(sw_ih_f.T), b16(sw_hh_f.T), (sb_ih_f + sb_hh_f).reshape(1, -1),
      b16(sw_ih_b.T), b16(sw_hh_b.T), (sb_ih_b + sb_hh_b).reshape(1, -1),
      b16(cw3[:, 0].transpose(1, 2, 0)), cb3.reshape(1, -1),
      b16(cw4[:, 0].transpose(1, 2, 0)), cb4.reshape(1, -1),
      b16(cw5[:, 0].transpose(1, 2, 0)), cb5.reshape(1, -1),
      jnp.eye(S, k=1, dtype=jnp.bfloat16),
      b16(dw1T[0:2 * SH]), b16(dw1T[2 * SH:4 * SH]),
      b16(dw1T[4 * SH:4 * SH + NF]), b16(dw1T[4 * SH + NF:4 * SH + 2 * NF]),
      b16(dw1T[4 * SH + 2 * NF:4 * SH + 3 * NF]),
      db1.reshape(1, -1), b16(dw2.T), db2.reshape(1, -1),
  )


def kernel(x, emb, ww_ih_f, ww_hh_f, wb_ih_f, wb_hh_f, ww_ih_b, ww_hh_b,
           wb_ih_b, wb_hh_b, sw_ih_f, sw_hh_f, sb_ih_f, sb_hh_f, sw_ih_b,
           sw_hh_b, sb_ih_b, sb_hh_b, cw3, cb3, cw4, cb4, cw5, cb5,
           dw1, db1, dw2, db2):
  xv = x.astype(jnp.int32)
  idx = xv.T.reshape(-1)
  wf = _sc_gather(emb, idx)
  wf3 = wf.reshape(L, S, EMB)
  dense_args = _prep_dense_args(
      ww_ih_f, ww_hh_f, wb_ih_f, wb_hh_f, ww_ih_b, ww_hh_b, wb_ih_b,
      wb_hh_b, sw_ih_f, sw_hh_f, sb_ih_f, sb_hh_f, sw_ih_b, sw_hh_b,
      sb_ih_b, sb_hh_b, cw3, cb3, cw4, cb4, cw5, cb5, dw1, db1, dw2, db2)
  return _dense_forward(wf3, xv, dense_args)


# speedup vs baseline: 1.2012x; 1.1019x over previous
import functools

import jax
import jax.numpy as jnp
from jax.experimental import pallas as pl
from jax.experimental.pallas import tpu as pltpu
from jax.experimental.pallas import tpu_sc as plsc

S = 128
L = 100
EMB = 256
WH = 256
SH = 256
NF = 100

_SC_CORES = 2
_SC_SUBCORES = 16
_SC_WORKERS = _SC_CORES * _SC_SUBCORES


def _sc_gather(table, idx):
  B = idx.shape[0]
  D = table.shape[1]
  b_per_w = B // _SC_WORKERS
  mesh = plsc.VectorSubcoreMesh(core_axis_name="c", subcore_axis_name="s")

  @functools.partial(
      pl.kernel,
      out_type=jax.ShapeDtypeStruct((B, D), jnp.float32),
      mesh=mesh,
      scratch_types=[
          pltpu.VMEM((b_per_w,), jnp.int32),
          pltpu.VMEM((b_per_w, D), jnp.float32),
          pltpu.SemaphoreType.DMA,
      ],
  )
  def gather_kernel(table_hbm, idx_hbm, out_hbm, idx_v, rows_v, sem):
    wid = jax.lax.axis_index("s") * _SC_CORES + jax.lax.axis_index("c")
    base = wid * b_per_w
    pltpu.sync_copy(idx_hbm.at[pl.ds(base, b_per_w)], idx_v)
    pltpu.async_copy(table_hbm.at[idx_v], rows_v, sem).wait()
    pltpu.sync_copy(rows_v, out_hbm.at[pl.ds(base, b_per_w)])

  return gather_kernel(table, idx)


def _lstm_gates(g, c):
  i = g[:, 0 * WH:1 * WH]
  f = g[:, 1 * WH:2 * WH]
  gg = g[:, 2 * WH:3 * WH]
  o = g[:, 3 * WH:4 * WH]
  c_new = jax.nn.sigmoid(f) * c + jax.nn.sigmoid(i) * jnp.tanh(gg)
  h_new = jax.nn.sigmoid(o) * jnp.tanh(c_new)
  return h_new, c_new


def _dot(a, b):
  return jnp.dot(a, b, preferred_element_type=jnp.float32)


_CB = 10
_NB = L // _CB


def _dense_body(wf_ref, x_ref, wih_f_ref, whh_f_ref, wbf_ref, wih_b_ref,
                whh_b_ref, wbb_ref, swihT_f_ref, swhhT_f_ref, sbf_ref,
                swihT_b_ref, swhhT_b_ref, sbb_ref, cw3t_ref, cb3_ref,
                cw4t_ref, cb4_ref, cw5t_ref, cb5_ref, shift_ref, dA_ref,
                dB_ref, dC3_ref, dC4_ref, dC5_ref, db1_ref, dw2T_ref,
                db2_ref, out_ref, buf_ref, sem_ref, xpf_ref, xpb_ref,
                sof_ref, sob_ref):
  xv = x_ref[...]
  seq_len = jnp.sum(jnp.sign(xv), axis=1, keepdims=True)

  wih_f = wih_f_ref[...]
  whh_f = whh_f_ref[...]
  wih_b = wih_b_ref[...]
  whh_b = whh_b_ref[...]
  wbf = wbf_ref[...]
  wbb = wbb_ref[...]

  def wf_copy(chunk, slot):
    return pltpu.make_async_copy(
        wf_ref.at[pl.ds(chunk * _CB, _CB)], buf_ref.at[slot],
        sem_ref.at[slot])

  wf_copy(0, 0).start()
  wf_copy(_NB - 1, 2).start()
  wf_copy(1, 1).start()
  wf_copy(_NB - 2, 3).start()

  def word_chunk(c, carry):
    slot_f = jax.lax.rem(c, 2)
    slot_b = 2 + slot_f
    wf_copy(c, slot_f).wait()
    wf_copy(_NB - 1 - c, slot_b).wait()

    def gate_step(i, carry2):
      for u in range(2):
        hf, cf, af, hb, cb, ab = carry2
        tl = 2 * i + u
        t = c * _CB + tl
        tb = L - 1 - t
        xt_f = buf_ref[slot_f, tl].astype(jnp.bfloat16)
        xt_b = buf_ref[slot_b, _CB - 1 - tl].astype(jnp.bfloat16)
        gf = _dot(xt_f, wih_f) + _dot(hf.astype(jnp.bfloat16), whh_f) + wbf
        gb = _dot(xt_b, wih_b) + _dot(hb.astype(jnp.bfloat16), whh_b) + wbb
        hf, cf = _lstm_gates(gf, cf)
        hb, cb = _lstm_gates(gb, cb)
        mf = jnp.where(t < seq_len, 1.0, 0.0)
        mb = jnp.where(tb < seq_len, 1.0, 0.0)
        carry2 = (hf, cf, af + hf * mf, hb, cb, ab + hb * mb)
      return carry2

    carry = jax.lax.fori_loop(0, _CB // 2, gate_step, carry)

    @pl.when(c + 2 < _NB)
    def _():
      wf_copy(c + 2, slot_f).start()
      wf_copy(_NB - 3 - c, slot_b).start()

    return carry

  z = jnp.zeros((S, WH), jnp.float32)
  _, _, af, _, _, ab = jax.lax.fori_loop(
      0, _NB, word_chunk, (z, z, z, z, z, z))

  denom = jnp.maximum(seq_len.astype(jnp.float32), 1.0)
  sent_f = af / denom
  sent_b = ab / denom
  sent = jnp.concatenate([sent_f, sent_b], axis=1).astype(jnp.bfloat16)

  xpf_ref[...] = _dot(sent, swihT_f_ref[...]) + sbf_ref[...]
  xpb_ref[...] = _dot(sent, swihT_b_ref[...]) + sbb_ref[...]
  swhhT_f = swhhT_f_ref[...]
  swhhT_b = swhhT_b_ref[...]

  def sent_step(i, carry):
    for u in range(2):
      hf, cf, hb, cb = carry
      t = 2 * i + u
      tb = S - 1 - t
      gf = xpf_ref[pl.ds(t, 1), :] + _dot(hf.astype(jnp.bfloat16), swhhT_f)
      gb = xpb_ref[pl.ds(tb, 1), :] + _dot(hb.astype(jnp.bfloat16), swhhT_b)
      hf, cf = _lstm_gates(gf, cf)
      hb, cb = _lstm_gates(gb, cb)
      sof_ref[pl.ds(t, 1), :] = hf
      sob_ref[pl.ds(tb, 1), :] = hb
      carry = (hf, cf, hb, cb)
    return carry

  z1 = jnp.zeros((1, SH), jnp.float32)
  jax.lax.fori_loop(0, S // 2, sent_step, (z1, z1, z1, z1))

  so = jnp.concatenate([sof_ref[...], sob_ref[...]], axis=1)
  doc = jnp.mean(so, axis=0, keepdims=True)
  so16 = so.astype(jnp.bfloat16)

  shift = shift_ref[...]
  rows = jax.lax.broadcasted_iota(jnp.int32, (S, NF), 0)

  def conv_pool(wt_ref, k, bias):
    acc = _dot(so16, wt_ref[k - 1])
    for j in range(k - 2, -1, -1):
      acc = _dot(so16, wt_ref[j]) + _dot(shift, acc.astype(jnp.bfloat16))
    out = jax.nn.relu(acc + bias)
    out = jnp.where(rows < S - k + 1, out, 0.0)
    return jnp.max(out, axis=0, keepdims=True)

  l3 = conv_pool(cw3t_ref, 3, cb3_ref[...])
  l4 = conv_pool(cw4t_ref, 4, cb4_ref[...])
  l5 = conv_pool(cw5t_ref, 5, cb5_ref[...])

  h = jnp.tanh(_dot(so16, dA_ref[...]) +
               _dot(doc.astype(jnp.bfloat16), dB_ref[...]) +
               _dot(l3.astype(jnp.bfloat16), dC3_ref[...]) +
               _dot(l4.astype(jnp.bfloat16), dC4_ref[...]) +
               _dot(l5.astype(jnp.bfloat16), dC5_ref[...]) + db1_ref[...])
  out_ref[...] = jax.nn.sigmoid(
      _dot(h.astype(jnp.bfloat16), dw2T_ref[...]) + db2_ref[...])


def _dense_forward(wf3, xv, dense_args, interpret=False):
  return pl.pallas_call(
      _dense_body,
      out_shape=jax.ShapeDtypeStruct((S, 1), jnp.float32),
      in_specs=[pl.BlockSpec(memory_space=pltpu.MemorySpace.HBM)] +
               [pl.BlockSpec(memory_space=pltpu.MemorySpace.VMEM)] *
               (1 + len(dense_args)),
      scratch_shapes=[
          pltpu.VMEM((4, _CB, S, EMB), jnp.float32),
          pltpu.SemaphoreType.DMA((4,)),
          pltpu.VMEM((S, 4 * SH), jnp.float32),
          pltpu.VMEM((S, 4 * SH), jnp.float32),
          pltpu.VMEM((S, SH), jnp.float32),
          pltpu.VMEM((S, SH), jnp.float32),
      ],
      compiler_params=pltpu.CompilerParams(
          vmem_limit_bytes=110 * 1024 * 1024),
      interpret=interpret,
  )(wf3, xv, *dense_args)


def _prep_dense_args(ww_ih_f, ww_hh_f, wb_ih_f, wb_hh_f, ww_ih_b, ww_hh_b,
                     wb_ih_b, wb_hh_b, sw_ih_f, sw_hh_f, sb_ih_f, sb_hh_f,
                     sw_ih_b, sw_hh_b, sb_ih_b, sb_hh_b, cw3, cb3, cw4, cb4,
                     cw5, cb5, dw1, db1, dw2, db2):
  dw1T = dw1.T
  b16 = lambda a: a.astype(jnp.bfloat16)
  return (
      b16(ww_ih_f.T), b16(ww_hh_f.T), (wb_ih_f + wb_hh_f).reshape(1, -1),
      b16(ww_ih_b.T), b16(ww_hh_b.T), (wb_ih_b + wb_hh_b).reshape(1, -1),
      b16(sw_ih_f.T), b16(sw_hh_f.T), (sb_ih_f + sb_hh_f).reshape(1, -1),
      b16(sw_ih_b.T), b16(sw_hh_b.T), (sb_ih_b + sb_hh_b).reshape(1, -1),
      b16(cw3[:, 0].transpose(1, 2, 0)), cb3.reshape(1, -1),
      b16(cw4[:, 0].transpose(1, 2, 0)), cb4.reshape(1, -1),
      b16(cw5[:, 0].transpose(1, 2, 0)), cb5.reshape(1, -1),
      jnp.eye(S, k=1, dtype=jnp.bfloat16),
      b16(dw1T[0:2 * SH]), b16(dw1T[2 * SH:4 * SH]),
      b16(dw1T[4 * SH:4 * SH + NF]), b16(dw1T[4 * SH + NF:4 * SH + 2 * NF]),
      b16(dw1T[4 * SH + 2 * NF:4 * SH + 3 * NF]),
      db1.reshape(1, -1), b16(dw2.T), db2.reshape(1, -1),
  )


def kernel(x, emb, ww_ih_f, ww_hh_f, wb_ih_f, wb_hh_f, ww_ih_b, ww_hh_b,
           wb_ih_b, wb_hh_b, sw_ih_f, sw_hh_f, sb_ih_f, sb_hh_f, sw_ih_b,
           sw_hh_b, sb_ih_b, sb_hh_b, cw3, cb3, cw4, cb4, cw5, cb5,
           dw1, db1, dw2, db2):
  xv = x.astype(jnp.int32)
  idx = xv.T.reshape(-1)
  wf = _sc_gather(emb, idx)
  wf3 = wf.reshape(L, S, EMB)
  dense_args = _prep_dense_args(
      ww_ih_f, ww_hh_f, wb_ih_f, wb_hh_f, ww_ih_b, ww_hh_b, wb_ih_b,
      wb_hh_b, sw_ih_f, sw_hh_f, sb_ih_f, sb_hh_f, sw_ih_b, sw_hh_b,
      sb_ih_b, sb_hh_b, cw3, cb3, cw4, cb4, cw5, cb5, dw1, db1, dw2, db2)
  return _dense_forward(wf3, xv, dense_args)


# speedup vs baseline: 1.2419x; 1.0339x over previous
import functools

import jax
import jax.numpy as jnp
from jax.experimental import pallas as pl
from jax.experimental.pallas import tpu as pltpu
from jax.experimental.pallas import tpu_sc as plsc

S = 128
L = 100
EMB = 256
WH = 256
SH = 256
NF = 100

_SC_CORES = 2
_SC_SUBCORES = 16
_SC_WORKERS = _SC_CORES * _SC_SUBCORES


def _sc_gather(table, idx):
  B = idx.shape[0]
  D = table.shape[1]
  b_per_w = B // _SC_WORKERS
  mesh = plsc.VectorSubcoreMesh(core_axis_name="c", subcore_axis_name="s")

  @functools.partial(
      pl.kernel,
      out_type=jax.ShapeDtypeStruct((B, D), jnp.float32),
      mesh=mesh,
      scratch_types=[
          pltpu.VMEM((b_per_w,), jnp.int32),
          pltpu.VMEM((b_per_w, D), jnp.float32),
          pltpu.SemaphoreType.DMA,
      ],
  )
  def gather_kernel(table_hbm, idx_hbm, out_hbm, idx_v, rows_v, sem):
    wid = jax.lax.axis_index("s") * _SC_CORES + jax.lax.axis_index("c")
    base = wid * b_per_w
    pltpu.sync_copy(idx_hbm.at[pl.ds(base, b_per_w)], idx_v)
    pltpu.async_copy(table_hbm.at[idx_v], rows_v, sem).wait()
    pltpu.sync_copy(rows_v, out_hbm.at[pl.ds(base, b_per_w)])

  return gather_kernel(table, idx)


def _lstm_gates(g, c):
  i = g[:, 0 * WH:1 * WH]
  f = g[:, 1 * WH:2 * WH]
  gg = g[:, 2 * WH:3 * WH]
  o = g[:, 3 * WH:4 * WH]
  c_new = jax.nn.sigmoid(f) * c + jax.nn.sigmoid(i) * jnp.tanh(gg)
  h_new = jax.nn.sigmoid(o) * jnp.tanh(c_new)
  return h_new, c_new


def _dot(a, b):
  return jnp.dot(a, b, preferred_element_type=jnp.float32)


_CB = 20
_NB = L // _CB


def _dense_body(wf_ref, x_ref, wih_f_ref, whh_f_ref, wbf_ref, wih_b_ref,
                whh_b_ref, wbb_ref, swihT_f_ref, swhhT_f_ref, sbf_ref,
                swihT_b_ref, swhhT_b_ref, sbb_ref, cw3t_ref, cb3_ref,
                cw4t_ref, cb4_ref, cw5t_ref, cb5_ref, shift_ref, dA_ref,
                dB_ref, dC3_ref, dC4_ref, dC5_ref, db1_ref, dw2T_ref,
                db2_ref, out_ref, buf_ref, sem_ref, xpf_ref, xpb_ref,
                sof_ref, sob_ref):
  xv = x_ref[...]
  seq_len = jnp.sum(jnp.sign(xv), axis=1, keepdims=True)

  wih_f = wih_f_ref[...]
  whh_f = whh_f_ref[...]
  wih_b = wih_b_ref[...]
  whh_b = whh_b_ref[...]
  wbf = wbf_ref[...]
  wbb = wbb_ref[...]

  def wf_copy(chunk, slot):
    return pltpu.make_async_copy(
        wf_ref.at[pl.ds(chunk * _CB, _CB)], buf_ref.at[slot],
        sem_ref.at[slot])

  wf_copy(0, 0).start()
  wf_copy(_NB - 1, 2).start()
  wf_copy(1, 1).start()
  wf_copy(_NB - 2, 3).start()

  def word_chunk(c, carry):
    slot_f = jax.lax.rem(c, 2)
    slot_b = 2 + slot_f
    wf_copy(c, slot_f).wait()
    wf_copy(_NB - 1 - c, slot_b).wait()

    def gate_step(i, carry2):
      for u in range(2):
        hf, cf, af, hb, cb, ab = carry2
        tl = 2 * i + u
        t = c * _CB + tl
        tb = L - 1 - t
        xt_f = buf_ref[slot_f, tl].astype(jnp.bfloat16)
        xt_b = buf_ref[slot_b, _CB - 1 - tl].astype(jnp.bfloat16)
        gf = _dot(xt_f, wih_f) + _dot(hf.astype(jnp.bfloat16), whh_f) + wbf
        gb = _dot(xt_b, wih_b) + _dot(hb.astype(jnp.bfloat16), whh_b) + wbb
        hf, cf = _lstm_gates(gf, cf)
        hb, cb = _lstm_gates(gb, cb)
        mf = jnp.where(t < seq_len, 1.0, 0.0)
        mb = jnp.where(tb < seq_len, 1.0, 0.0)
        carry2 = (hf, cf, af + hf * mf, hb, cb, ab + hb * mb)
      return carry2

    carry = jax.lax.fori_loop(0, _CB // 2, gate_step, carry)

    @pl.when(c + 2 < _NB)
    def _():
      wf_copy(c + 2, slot_f).start()
      wf_copy(_NB - 3 - c, slot_b).start()

    return carry

  z = jnp.zeros((S, WH), jnp.float32)
  _, _, af, _, _, ab = jax.lax.fori_loop(
      0, _NB, word_chunk, (z, z, z, z, z, z))

  denom = jnp.maximum(seq_len.astype(jnp.float32), 1.0)
  sent_f = af / denom
  sent_b = ab / denom
  sent = jnp.concatenate([sent_f, sent_b], axis=1).astype(jnp.bfloat16)

  xpf_ref[...] = _dot(sent, swihT_f_ref[...]) + sbf_ref[...]
  xpb_ref[...] = _dot(sent, swihT_b_ref[...]) + sbb_ref[...]
  swhhT_f = swhhT_f_ref[...]
  swhhT_b = swhhT_b_ref[...]

  def sent_step(i, carry):
    for u in range(4):
      hf, cf, hb, cb = carry
      t = 4 * i + u
      tb = S - 1 - t
      gf = xpf_ref[pl.ds(t, 1), :] + _dot(hf.astype(jnp.bfloat16), swhhT_f)
      gb = xpb_ref[pl.ds(tb, 1), :] + _dot(hb.astype(jnp.bfloat16), swhhT_b)
      hf, cf = _lstm_gates(gf, cf)
      hb, cb = _lstm_gates(gb, cb)
      sof_ref[pl.ds(t, 1), :] = hf
      sob_ref[pl.ds(tb, 1), :] = hb
      carry = (hf, cf, hb, cb)
    return carry

  z1 = jnp.zeros((1, SH), jnp.float32)
  jax.lax.fori_loop(0, S // 4, sent_step, (z1, z1, z1, z1))

  so = jnp.concatenate([sof_ref[...], sob_ref[...]], axis=1)
  doc = jnp.mean(so, axis=0, keepdims=True)
  so16 = so.astype(jnp.bfloat16)

  shift = shift_ref[...]
  rows = jax.lax.broadcasted_iota(jnp.int32, (S, NF), 0)

  def conv_pool(wt_ref, k, bias):
    acc = _dot(so16, wt_ref[k - 1])
    for j in range(k - 2, -1, -1):
      acc = _dot(so16, wt_ref[j]) + _dot(shift, acc.astype(jnp.bfloat16))
    out = jax.nn.relu(acc + bias)
    out = jnp.where(rows < S - k + 1, out, 0.0)
    return jnp.max(out, axis=0, keepdims=True)

  l3 = conv_pool(cw3t_ref, 3, cb3_ref[...])
  l4 = conv_pool(cw4t_ref, 4, cb4_ref[...])
  l5 = conv_pool(cw5t_ref, 5, cb5_ref[...])

  h = jnp.tanh(_dot(so16, dA_ref[...]) +
               _dot(doc.astype(jnp.bfloat16), dB_ref[...]) +
               _dot(l3.astype(jnp.bfloat16), dC3_ref[...]) +
               _dot(l4.astype(jnp.bfloat16), dC4_ref[...]) +
               _dot(l5.astype(jnp.bfloat16), dC5_ref[...]) + db1_ref[...])
  out_ref[...] = jax.nn.sigmoid(
      _dot(h.astype(jnp.bfloat16), dw2T_ref[...]) + db2_ref[...])


def _dense_forward(wf3, xv, dense_args, interpret=False):
  return pl.pallas_call(
      _dense_body,
      out_shape=jax.ShapeDtypeStruct((S, 1), jnp.float32),
      in_specs=[pl.BlockSpec(memory_space=pltpu.MemorySpace.HBM)] +
               [pl.BlockSpec(memory_space=pltpu.MemorySpace.VMEM)] *
               (1 + len(dense_args)),
      scratch_shapes=[
          pltpu.VMEM((4, _CB, S, EMB), jnp.float32),
          pltpu.SemaphoreType.DMA((4,)),
          pltpu.VMEM((S, 4 * SH), jnp.float32),
          pltpu.VMEM((S, 4 * SH), jnp.float32),
          pltpu.VMEM((S, SH), jnp.float32),
          pltpu.VMEM((S, SH), jnp.float32),
      ],
      compiler_params=pltpu.CompilerParams(
          vmem_limit_bytes=110 * 1024 * 1024),
      interpret=interpret,
  )(wf3, xv, *dense_args)


def _prep_dense_args(ww_ih_f, ww_hh_f, wb_ih_f, wb_hh_f, ww_ih_b, ww_hh_b,
                     wb_ih_b, wb_hh_b, sw_ih_f, sw_hh_f, sb_ih_f, sb_hh_f,
                     sw_ih_b, sw_hh_b, sb_ih_b, sb_hh_b, cw3, cb3, cw4, cb4,
                     cw5, cb5, dw1, db1, dw2, db2):
  dw1T = dw1.T
  b16 = lambda a: a.astype(jnp.bfloat16)
  return (
      b16(ww_ih_f.T), b16(ww_hh_f.T), (wb_ih_f + wb_hh_f).reshape(1, -1),
      b16(ww_ih_b.T), b16(ww_hh_b.T), (wb_ih_b + wb_hh_b).reshape(1, -1),
      b16(sw_ih_f.T), b16(sw_hh_f.T), (sb_ih_f + sb_hh_f).reshape(1, -1),
      b16(sw_ih_b.T), b16(sw_hh_b.T), (sb_ih_b + sb_hh_b).reshape(1, -1),
      b16(cw3[:, 0].transpose(1, 2, 0)), cb3.reshape(1, -1),
      b16(cw4[:, 0].transpose(1, 2, 0)), cb4.reshape(1, -1),
      b16(cw5[:, 0].transpose(1, 2, 0)), cb5.reshape(1, -1),
      jnp.eye(S, k=1, dtype=jnp.bfloat16),
      b16(dw1T[0:2 * SH]), b16(dw1T[2 * SH:4 * SH]),
      b16(dw1T[4 * SH:4 * SH + NF]), b16(dw1T[4 * SH + NF:4 * SH + 2 * NF]),
      b16(dw1T[4 * SH + 2 * NF:4 * SH + 3 * NF]),
      db1.reshape(1, -1), b16(dw2.T), db2.reshape(1, -1),
  )


def kernel(x, emb, ww_ih_f, ww_hh_f, wb_ih_f, wb_hh_f, ww_ih_b, ww_hh_b,
           wb_ih_b, wb_hh_b, sw_ih_f, sw_hh_f, sb_ih_f, sb_hh_f, sw_ih_b,
           sw_hh_b, sb_ih_b, sb_hh_b, cw3, cb3, cw4, cb4, cw5, cb5,
           dw1, db1, dw2, db2):
  xv = x.astype(jnp.int32)
  idx = xv.T.reshape(-1)
  wf = _sc_gather(emb, idx)
  wf3 = wf.reshape(L, S, EMB)
  dense_args = _prep_dense_args(
      ww_ih_f, ww_hh_f, wb_ih_f, wb_hh_f, ww_ih_b, ww_hh_b, wb_ih_b,
      wb_hh_b, sw_ih_f, sw_hh_f, sb_ih_f, sb_hh_f, sw_ih_b, sw_hh_b,
      sb_ih_b, sb_hh_b, cw3, cb3, cw4, cb4, cw5, cb5, dw1, db1, dw2, db2)
  return _dense_forward(wf3, xv, dense_args)


# speedup vs baseline: 1.3109x; 1.0556x over previous
import functools

import jax
import jax.numpy as jnp
from jax.experimental import pallas as pl
from jax.experimental.pallas import tpu as pltpu
from jax.experimental.pallas import tpu_sc as plsc

S = 128
L = 100
EMB = 256
WH = 256
SH = 256
NF = 100

_SC_CORES = 2
_SC_SUBCORES = 16
_SC_WORKERS = _SC_CORES * _SC_SUBCORES


def _sc_gather(table, idx):
  B = idx.shape[0]
  D = table.shape[1]
  b_per_w = B // _SC_WORKERS
  mesh = plsc.VectorSubcoreMesh(core_axis_name="c", subcore_axis_name="s")

  @functools.partial(
      pl.kernel,
      out_type=jax.ShapeDtypeStruct((B, D), jnp.float32),
      mesh=mesh,
      scratch_types=[
          pltpu.VMEM((b_per_w,), jnp.int32),
          pltpu.VMEM((b_per_w, D), jnp.float32),
          pltpu.SemaphoreType.DMA,
      ],
  )
  def gather_kernel(table_hbm, idx_hbm, out_hbm, idx_v, rows_v, sem):
    wid = jax.lax.axis_index("s") * _SC_CORES + jax.lax.axis_index("c")
    base = wid * b_per_w
    pltpu.sync_copy(idx_hbm.at[pl.ds(base, b_per_w)], idx_v)
    pltpu.async_copy(table_hbm.at[idx_v], rows_v, sem).wait()
    pltpu.sync_copy(rows_v, out_hbm.at[pl.ds(base, b_per_w)])

  return gather_kernel(table, idx)


def _lstm_gates(g, c):
  i = g[:, 0 * WH:1 * WH]
  f = g[:, 1 * WH:2 * WH]
  gg = g[:, 2 * WH:3 * WH]
  o = g[:, 3 * WH:4 * WH]
  c_new = jax.nn.sigmoid(f) * c + jax.nn.sigmoid(i) * jnp.tanh(gg)
  h_new = jax.nn.sigmoid(o) * jnp.tanh(c_new)
  return h_new, c_new


def _dot(a, b):
  return jnp.dot(a, b, preferred_element_type=jnp.float32)


_CB = 20
_NB = L // _CB


def _dense_body(wf_ref, x_ref, wih_f_ref, whh_f_ref, wbf_ref, wih_b_ref,
                whh_b_ref, wbb_ref, swihT_f_ref, swhhT_f_ref, sbf_ref,
                swihT_b_ref, swhhT_b_ref, sbb_ref, cw3t_ref, cb3_ref,
                cw4t_ref, cb4_ref, cw5t_ref, cb5_ref, shift_ref, dA_ref,
                dB_ref, dC3_ref, dC4_ref, dC5_ref, db1_ref, dw2T_ref,
                db2_ref, out_ref, buf_ref, sem_ref, xpf_ref, xpb_ref,
                sof_ref, sob_ref):
  xv = x_ref[...]
  seq_len = jnp.sum(jnp.sign(xv), axis=1, keepdims=True)

  wih_f = wih_f_ref[...]
  whh_f = whh_f_ref[...]
  wih_b = wih_b_ref[...]
  whh_b = whh_b_ref[...]
  wbf = wbf_ref[...]
  wbb = wbb_ref[...]

  def wf_copy(chunk, slot):
    return pltpu.make_async_copy(
        wf_ref.at[pl.ds(chunk * _CB, _CB)], buf_ref.at[slot],
        sem_ref.at[slot])

  wf_copy(0, 0).start()
  wf_copy(_NB - 1, 2).start()
  wf_copy(1, 1).start()
  wf_copy(_NB - 2, 3).start()

  def word_chunk(c, carry):
    slot_f = jax.lax.rem(c, 2)
    slot_b = 2 + slot_f
    wf_copy(c, slot_f).wait()
    wf_copy(_NB - 1 - c, slot_b).wait()

    def gate_step(i, carry2):
      for u in range(4):
        hf, cf, af, hb, cb, ab = carry2
        tl = 4 * i + u
        t = c * _CB + tl
        tb = L - 1 - t
        xt_f = buf_ref[slot_f, tl].astype(jnp.bfloat16)
        xt_b = buf_ref[slot_b, _CB - 1 - tl].astype(jnp.bfloat16)
        gf = _dot(xt_f, wih_f) + _dot(hf.astype(jnp.bfloat16), whh_f) + wbf
        gb = _dot(xt_b, wih_b) + _dot(hb.astype(jnp.bfloat16), whh_b) + wbb
        hf, cf = _lstm_gates(gf, cf)
        hb, cb = _lstm_gates(gb, cb)
        mf = jnp.where(t < seq_len, 1.0, 0.0)
        mb = jnp.where(tb < seq_len, 1.0, 0.0)
        carry2 = (hf, cf, af + hf * mf, hb, cb, ab + hb * mb)
      return carry2

    carry = jax.lax.fori_loop(0, _CB // 4, gate_step, carry)

    @pl.when(c + 2 < _NB)
    def _():
      wf_copy(c + 2, slot_f).start()
      wf_copy(_NB - 3 - c, slot_b).start()

    return carry

  z = jnp.zeros((S, WH), jnp.float32)
  _, _, af, _, _, ab = jax.lax.fori_loop(
      0, _NB, word_chunk, (z, z, z, z, z, z))

  denom = jnp.maximum(seq_len.astype(jnp.float32), 1.0)
  sent_f = af / denom
  sent_b = ab / denom
  sent = jnp.concatenate([sent_f, sent_b], axis=1).astype(jnp.bfloat16)

  xpf_ref[...] = _dot(sent, swihT_f_ref[...]) + sbf_ref[...]
  xpb_ref[...] = _dot(sent, swihT_b_ref[...]) + sbb_ref[...]
  swhhT_f = swhhT_f_ref[...]
  swhhT_b = swhhT_b_ref[...]

  def sent_step(i, carry):
    for u in range(4):
      hf, cf, hb, cb = carry
      t = 4 * i + u
      tb = S - 1 - t
      gf = xpf_ref[pl.ds(t, 1), :] + _dot(hf.astype(jnp.bfloat16), swhhT_f)
      gb = xpb_ref[pl.ds(tb, 1), :] + _dot(hb.astype(jnp.bfloat16), swhhT_b)
      hf, cf = _lstm_gates(gf, cf)
      hb, cb = _lstm_gates(gb, cb)
      sof_ref[pl.ds(t, 1), :] = hf
      sob_ref[pl.ds(tb, 1), :] = hb
      carry = (hf, cf, hb, cb)
    return carry

  z1 = jnp.zeros((1, SH), jnp.float32)
  jax.lax.fori_loop(0, S // 4, sent_step, (z1, z1, z1, z1))

  so = jnp.concatenate([sof_ref[...], sob_ref[...]], axis=1)
  doc = jnp.mean(so, axis=0, keepdims=True)
  so16 = so.astype(jnp.bfloat16)

  shift = shift_ref[...]
  rows = jax.lax.broadcasted_iota(jnp.int32, (S, NF), 0)

  def conv_pool(wt_ref, k, bias):
    acc = _dot(so16, wt_ref[k - 1])
    for j in range(k - 2, -1, -1):
      acc = _dot(so16, wt_ref[j]) + _dot(shift, acc.astype(jnp.bfloat16))
    out = jax.nn.relu(acc + bias)
    out = jnp.where(rows < S - k + 1, out, 0.0)
    return jnp.max(out, axis=0, keepdims=True)

  l3 = conv_pool(cw3t_ref, 3, cb3_ref[...])
  l4 = conv_pool(cw4t_ref, 4, cb4_ref[...])
  l5 = conv_pool(cw5t_ref, 5, cb5_ref[...])

  h = jnp.tanh(_dot(so16, dA_ref[...]) +
               _dot(doc.astype(jnp.bfloat16), dB_ref[...]) +
               _dot(l3.astype(jnp.bfloat16), dC3_ref[...]) +
               _dot(l4.astype(jnp.bfloat16), dC4_ref[...]) +
               _dot(l5.astype(jnp.bfloat16), dC5_ref[...]) + db1_ref[...])
  out_ref[...] = jax.nn.sigmoid(
      _dot(h.astype(jnp.bfloat16), dw2T_ref[...]) + db2_ref[...])


def _dense_forward(wf3, xv, dense_args, interpret=False):
  return pl.pallas_call(
      _dense_body,
      out_shape=jax.ShapeDtypeStruct((S, 1), jnp.float32),
      in_specs=[pl.BlockSpec(memory_space=pltpu.MemorySpace.HBM)] +
               [pl.BlockSpec(memory_space=pltpu.MemorySpace.VMEM)] *
               (1 + len(dense_args)),
      scratch_shapes=[
          pltpu.VMEM((4, _CB, S, EMB), jnp.float32),
          pltpu.SemaphoreType.DMA((4,)),
          pltpu.VMEM((S, 4 * SH), jnp.float32),
          pltpu.VMEM((S, 4 * SH), jnp.float32),
          pltpu.VMEM((S, SH), jnp.float32),
          pltpu.VMEM((S, SH), jnp.float32),
      ],
      compiler_params=pltpu.CompilerParams(
          vmem_limit_bytes=110 * 1024 * 1024),
      interpret=interpret,
  )(wf3, xv, *dense_args)


def _prep_dense_args(ww_ih_f, ww_hh_f, wb_ih_f, wb_hh_f, ww_ih_b, ww_hh_b,
                     wb_ih_b, wb_hh_b, sw_ih_f, sw_hh_f, sb_ih_f, sb_hh_f,
                     sw_ih_b, sw_hh_b, sb_ih_b, sb_hh_b, cw3, cb3, cw4, cb4,
                     cw5, cb5, dw1, db1, dw2, db2):
  dw1T = dw1.T
  b16 = lambda a: a.astype(jnp.bfloat16)
  return (
      b16(ww_ih_f.T), b16(ww_hh_f.T), (wb_ih_f + wb_hh_f).reshape(1, -1),
      b16(ww_ih_b.T), b16(ww_hh_b.T), (wb_ih_b + wb_hh_b).reshape(1, -1),
      b16(sw_ih_f.T), b16(sw_hh_f.T), (sb_ih_f + sb_hh_f).reshape(1, -1),
      b16(sw_ih_b.T), b16(sw_hh_b.T), (sb_ih_b + sb_hh_b).reshape(1, -1),
      b16(cw3[:, 0].transpose(1, 2, 0)), cb3.reshape(1, -1),
      b16(cw4[:, 0].transpose(1, 2, 0)), cb4.reshape(1, -1),
      b16(cw5[:, 0].transpose(1, 2, 0)), cb5.reshape(1, -1),
      jnp.eye(S, k=1, dtype=jnp.bfloat16),
      b16(dw1T[0:2 * SH]), b16(dw1T[2 * SH:4 * SH]),
      b16(dw1T[4 * SH:4 * SH + NF]), b16(dw1T[4 * SH + NF:4 * SH + 2 * NF]),
      b16(dw1T[4 * SH + 2 * NF:4 * SH + 3 * NF]),
      db1.reshape(1, -1), b16(dw2.T), db2.reshape(1, -1),
  )


def kernel(x, emb, ww_ih_f, ww_hh_f, wb_ih_f, wb_hh_f, ww_ih_b, ww_hh_b,
           wb_ih_b, wb_hh_b, sw_ih_f, sw_hh_f, sb_ih_f, sb_hh_f, sw_ih_b,
           sw_hh_b, sb_ih_b, sb_hh_b, cw3, cb3, cw4, cb4, cw5, cb5,
           dw1, db1, dw2, db2):
  xv = x.astype(jnp.int32)
  idx = xv.T.reshape(-1)
  wf = _sc_gather(emb, idx)
  wf3 = wf.reshape(L, S, EMB)
  dense_args = _prep_dense_args(
      ww_ih_f, ww_hh_f, wb_ih_f, wb_hh_f, ww_ih_b, ww_hh_b, wb_ih_b,
      wb_hh_b, sw_ih_f, sw_hh_f, sb_ih_f, sb_hh_f, sw_ih_b, sw_hh_b,
      sb_ih_b, sb_hh_b, cw3, cb3, cw4, cb4, cw5, cb5, dw1, db1, dw2, db2)
  return _dense_forward(wf3, xv, dense_args)
